# baseline (speedup 1.0000x reference)
import functools

import numpy as np

import jax
import jax.numpy as jnp
from jax import lax
from jax.experimental import pallas as pl
from jax.experimental.pallas import tpu as pltpu
from jax.experimental.pallas import tpu_sc as plsc

B = 4096
S = 200
S1 = S + 1
D = 64
COND_DIM = 128

NC = 2
NS = 16
NW = NC * NS
BBLK = B // NW
GBUF = 8
L = 16
NBG = BBLK // L
ND = D // L

_d = [np.arange(j * L, (j + 1) * L, dtype=np.int32) for j in range(ND)]


def _dense_body(ceT_ref, wcT_ref, bc_ref, condT_ref):
    condT_ref[...] = (
        jnp.dot(wcT_ref[...], ceT_ref[...], preferred_element_type=jnp.float32)
        + bc_ref[...]
    )


def _dense_tc(condition_emb, W_cond, b_cond):
    return pl.pallas_call(
        _dense_body,
        out_shape=jax.ShapeDtypeStruct((D, B), jnp.float32),
    )(condition_emb.T, W_cond.T, b_cond.reshape(D, 1))


def _sc_gather(xT, t, W_t, b_t, emb_table):
    mesh = plsc.VectorSubcoreMesh(core_axis_name="c", subcore_axis_name="s")

    @functools.partial(
        pl.kernel,
        mesh=mesh,
        compiler_params=pltpu.CompilerParams(
            use_tc_tiling_on_sc=False, needs_layout_passes=False),
        out_type=jax.ShapeDtypeStruct((S1, D // 8, B // 128, 8, 128),
                                      jnp.float32),
        scratch_types=[
            pltpu.VMEM((S, BBLK), jnp.int32),
            pltpu.VMEM((GBUF, BBLK, D), jnp.float32),
            pltpu.VMEM((2, D // 8, 1, 8, BBLK + 1), jnp.float32),
            pltpu.VMEM((BBLK,), jnp.float32),
            pltpu.VMEM((D,), jnp.float32),
            pltpu.VMEM((D,), jnp.float32),
            [pltpu.SemaphoreType.DMA] * GBUF,
            [pltpu.SemaphoreType.DMA] * 2,
        ],
    )
    def k(xT_hbm, t_hbm, wt_hbm, bt_hbm, table_hbm, p_hbm,
          idxv, g, tbuf, tv, wv, btv, gsems, ssems):
        wid = lax.axis_index("s") * NC + lax.axis_index("c")
        col = wid * BBLK

        pltpu.sync_copy(xT_hbm.at[:, pl.ds(col, BBLK)], idxv)
        pltpu.sync_copy(t_hbm.at[pl.ds(col, BBLK)], tv)
        pltpu.sync_copy(wt_hbm, wv)
        pltpu.sync_copy(bt_hbm, btv)

        def tb_at(tk, d):
            return tbuf.at[tk, d // 8, 0, d % 8]

        def tb_used(tk):
            return tbuf.at[tk, :, :, :, pl.ds(0, BBLK)]

        for bg in range(NBG):
            tb16 = tv[pl.ds(bg * L, L)]
            for d in range(D):
                sel = jnp.full((L,), d % L, jnp.int32)
                w_d = wv[pl.ds((d // L) * L, L)].at[sel].get(
                    mode="promise_in_bounds")
                b_d = btv[pl.ds((d // L) * L, L)].at[sel].get(
                    mode="promise_in_bounds")
                tb_at(0, d)[pl.ds(bg * L, L)] = tb16 * w_d + b_d
        pltpu.sync_copy(tb_used(0), p_hbm.at[0, :, pl.ds(wid, 1)])

        def issue(sp, kk):
            pltpu.async_copy(table_hbm.at[idxv.at[sp]], g.at[kk], gsems[kk])

        for kk in range(GBUF):
            issue(kk, kk)

        iota = lax.iota(jnp.int32, L)
        DT_IDX = [(iota + j * L) // 8 for j in range(ND)]
        R_IDX = [(iota + j * L) & 7 for j in range(ND)]
        Z_IDX = iota * 0

        @pl.loop(0, S, step=GBUF)
        def body(i):
            for kk in range(GBUF):
                sp = i + kk
                tk = kk % 2
                pltpu.make_async_copy(
                    table_hbm.at[pl.ds(0, BBLK)], g.at[kk], gsems[kk]).wait()

                @pl.when(sp >= 2)
                def _():
                    pltpu.make_async_copy(
                        p_hbm.at[0, :, pl.ds(0, 1)],
                        tb_used(tk), ssems[tk]).wait()

                @pl.loop(0, BBLK, unroll=8)
                def row(b):
                    cvec = Z_IDX + b
                    vals = [g[kk, b, pl.ds(j * L, L)] for j in range(ND)]
                    for j in range(ND):
                        plsc.store_scatter(
                            tbuf.at[tk],
                            [DT_IDX[j], Z_IDX, R_IDX[j], cvec], vals[j])

                pltpu.async_copy(
                    tb_used(tk), p_hbm.at[1 + sp, :, pl.ds(wid, 1)],
                    ssems[tk])

                nsp = sp + GBUF

                @pl.when(nsp < S)
                def _():
                    issue(nsp, kk)

        for tk in range(2):
            pltpu.make_async_copy(
                p_hbm.at[0, :, pl.ds(0, 1)], tb_used(tk), ssems[tk]).wait()

    return k(xT, t, W_t, b_t, emb_table)


def kernel(x, t, condition_emb, emb_table, W_cond, b_cond, W_t, b_t):
    condT = _dense_tc(condition_emb, W_cond, b_cond)
    p5 = _sc_gather(x.T, t, W_t.reshape(D), b_t, emb_table)
    dit = p5.transpose(2, 4, 0, 1, 3).reshape(B, S1, D)
    return (dit, condT.T)


# speedup vs baseline: 1.1545x; 1.1545x over previous
import functools

import numpy as np

import jax
import jax.numpy as jnp
from jax import lax
from jax.experimental import pallas as pl
from jax.experimental.pallas import tpu as pltpu
from jax.experimental.pallas import tpu_sc as plsc

B = 4096
S = 200
S1 = S + 1
D = 64
COND_DIM = 128

NC = 2
NS = 16
NW = NC * NS
BBLK = B // NW
GBUF = 4
L = 16
NBG = BBLK // L
ND = D // L

_d = [np.arange(j * L, (j + 1) * L, dtype=np.int32) for j in range(ND)]


def _dense_body(ceT_ref, wcT_ref, bc_ref, condT_ref):
    condT_ref[...] = (
        jnp.dot(wcT_ref[...], ceT_ref[...], preferred_element_type=jnp.float32)
        + bc_ref[...]
    )


def _dense_tc(condition_emb, W_cond, b_cond):
    return pl.pallas_call(
        _dense_body,
        out_shape=jax.ShapeDtypeStruct((D, B), jnp.float32),
    )(condition_emb.T, W_cond.T, b_cond.reshape(D, 1))


def _sc_gather(xT, t, W_t, b_t, emb_table):
    mesh = plsc.VectorSubcoreMesh(core_axis_name="c", subcore_axis_name="s")

    @functools.partial(
        pl.kernel,
        mesh=mesh,
        compiler_params=pltpu.CompilerParams(
            use_tc_tiling_on_sc=False, needs_layout_passes=False),
        out_type=jax.ShapeDtypeStruct((S1, D // 8, B // 128, 8, 128),
                                      jnp.float32),
        scratch_types=[
            pltpu.VMEM((S, BBLK), jnp.int32),
            pltpu.VMEM((GBUF, BBLK, D), jnp.float32),
            pltpu.VMEM((2, D // 8, 1, 8, BBLK + 1), jnp.float32),
            pltpu.VMEM((BBLK,), jnp.float32),
            pltpu.VMEM((D,), jnp.float32),
            pltpu.VMEM((D,), jnp.float32),
            [pltpu.SemaphoreType.DMA] * GBUF,
            [pltpu.SemaphoreType.DMA] * 2,
        ],
    )
    def k(xT_hbm, t_hbm, wt_hbm, bt_hbm, table_hbm, p_hbm,
          idxv, g, tbuf, tv, wv, btv, gsems, ssems):
        wid = lax.axis_index("s") * NC + lax.axis_index("c")
        col = wid * BBLK

        pltpu.sync_copy(xT_hbm.at[:, pl.ds(col, BBLK)], idxv)
        pltpu.sync_copy(t_hbm.at[pl.ds(col, BBLK)], tv)
        pltpu.sync_copy(wt_hbm, wv)
        pltpu.sync_copy(bt_hbm, btv)

        def tb_at(tk, d):
            return tbuf.at[tk, d // 8, 0, d % 8]

        def tb_used(tk):
            return tbuf.at[tk, :, :, :, pl.ds(0, BBLK)]

        for bg in range(NBG):
            tb16 = tv[pl.ds(bg * L, L)]
            for d in range(D):
                sel = jnp.full((L,), d % L, jnp.int32)
                w_d = wv[pl.ds((d // L) * L, L)].at[sel].get(
                    mode="promise_in_bounds")
                b_d = btv[pl.ds((d // L) * L, L)].at[sel].get(
                    mode="promise_in_bounds")
                tb_at(0, d)[pl.ds(bg * L, L)] = tb16 * w_d + b_d
        pltpu.sync_copy(tb_used(0), p_hbm.at[0, :, pl.ds(wid, 1)])

        def issue(sp, kk):
            pltpu.async_copy(table_hbm.at[idxv.at[sp]], g.at[kk], gsems[kk])

        for kk in range(GBUF):
            issue(kk, kk)

        iota = lax.iota(jnp.int32, L)
        PRE = [((iota + j * L) // 8) * 1032 + ((iota + j * L) & 7) * 129
               for j in range(ND)]
        Z_IDX = iota * 0

        @pl.loop(0, S, step=GBUF)
        def body(i):
            for kk in range(GBUF):
                sp = i + kk
                tk = kk % 2
                pltpu.make_async_copy(
                    table_hbm.at[pl.ds(0, BBLK)], g.at[kk], gsems[kk]).wait()

                @pl.when(sp >= 2)
                def _():
                    pltpu.make_async_copy(
                        p_hbm.at[0, :, pl.ds(0, 1)],
                        tb_used(tk), ssems[tk]).wait()

                @pl.loop(0, BBLK, unroll=8)
                def row(b):
                    vals = [g[kk, b, pl.ds(j * L, L)] for j in range(ND)]
                    for j in range(ND):
                        plsc.store_scatter(
                            tbuf.at[tk],
                            [Z_IDX, Z_IDX, Z_IDX, PRE[j] + b], vals[j])

                pltpu.async_copy(
                    tb_used(tk), p_hbm.at[1 + sp, :, pl.ds(wid, 1)],
                    ssems[tk])

                nsp = sp + GBUF

                @pl.when(nsp < S)
                def _():
                    issue(nsp, kk)

        for tk in range(2):
            pltpu.make_async_copy(
                p_hbm.at[0, :, pl.ds(0, 1)], tb_used(tk), ssems[tk]).wait()

    return k(xT, t, W_t, b_t, emb_table)


def kernel(x, t, condition_emb, emb_table, W_cond, b_cond, W_t, b_t):
    condT = _dense_tc(condition_emb, W_cond, b_cond)
    p5 = _sc_gather(x.T, t, W_t.reshape(D), b_t, emb_table)
    dit = p5.transpose(2, 4, 0, 1, 3).reshape(B, S1, D)
    return (dit, condT.T)
